# baseline (speedup 1.0000x reference)
import functools
import numpy as np
import jax
import jax.numpy as jnp
from jax.experimental import pallas as pl

_EPS = 1e-5


def _cdiv(a, b):
    return -(-a // b)


def _rup(a, b):
    return _cdiv(a, b) * b


def _conv_bn_body(x_ref, w_ref, y_ref, st_ref, *, taps, BM, C, Wpx, Wvalid, Mout):
    n = pl.program_id(0)
    b = pl.program_id(1)
    base = b * BM
    co = w_ref.shape[2]
    acc = jnp.zeros((BM, co), jnp.float32)
    for t, (ph, off) in enumerate(taps):
        g8, s = off & ~7, off & 7
        xs = x_ref[0, pl.ds(base + g8, BM + 8), pl.ds(ph * C, C)][s:s + BM]
        acc = acc + jnp.dot(xs, w_ref[t], preferred_element_type=jnp.float32)
    y_ref[0] = acc
    pidx = base + jax.lax.broadcasted_iota(jnp.int32, (BM, 1), 0)
    valid = ((pidx % Wpx) < Wvalid) & (pidx < Mout)
    m = valid.astype(jnp.float32)
    s1 = jnp.sum(acc * m, axis=0, keepdims=True)
    s2 = jnp.sum(acc * acc * m, axis=0, keepdims=True)

    @pl.when((n == 0) & (b == 0))
    def _():
        st_ref[...] = jnp.zeros_like(st_ref)

    st_ref[0:1, :] += s1
    st_ref[1:2, :] += s2


def _conv_sig_body(x_ref, w_ref, bias_ref, y_ref, *, taps, BM, C):
    b = pl.program_id(1)
    base = b * BM
    co = w_ref.shape[2]
    acc = jnp.zeros((BM, co), jnp.float32)
    for t, (ph, off) in enumerate(taps):
        g8, s = off & ~7, off & 7
        xs = x_ref[0, pl.ds(base + g8, BM + 8), pl.ds(ph * C, C)][s:s + BM]
        acc = acc + jnp.dot(xs, w_ref[t], preferred_element_type=jnp.float32)
    y_ref[0] = jax.nn.sigmoid(acc + bias_ref[0:1, :])


def _bn_relu_body(y_ref, ab_ref, t_ref):
    a = ab_ref[0:1, :]
    b = ab_ref[1:2, :]
    t_ref[0] = jnp.maximum(y_ref[0] * a + b, 0.0)


def _conv_call(xf, w9, nblk, BM, body):
    N, R, L = xf.shape
    C, Co = w9.shape[1], w9.shape[2]
    MP = nblk * BM
    return pl.pallas_call(
        body,
        grid=(N, nblk),
        in_specs=[
            pl.BlockSpec((1, R, L), lambda n, b: (n, 0, 0)),
            pl.BlockSpec((9, C, Co), lambda n, b: (0, 0, 0)),
        ],
        out_specs=[
            pl.BlockSpec((1, BM, Co), lambda n, b: (n, b, 0)),
            pl.BlockSpec((8, Co), lambda n, b: (0, 0)),
        ],
        out_shape=[
            jax.ShapeDtypeStruct((N, MP, Co), jnp.float32),
            jax.ShapeDtypeStruct((8, Co), jnp.float32),
        ],
    )(xf, w9)


def _enc_conv(act, w9, BM):
    N, H, W, C = act.shape
    Hp, Wp = H + 2, W + 2
    H2, W2 = Hp // 2, Wp // 2
    Ho, Wo = H // 2, W // 2
    Mout = Ho * W2
    BM = min(BM, _rup(Mout, 8))
    nblk = _cdiv(Mout, BM)
    R = nblk * BM + W2 + 9
    xp = jnp.pad(act, ((0, 0), (1, 1), (1, 1), (0, 0)))
    ph = xp.reshape(N, H2, 2, W2, 2, C).transpose(0, 1, 3, 2, 4, 5)
    ph = ph.reshape(N, H2 * W2, 4 * C)
    ph = jnp.pad(ph, ((0, 0), (0, R - H2 * W2), (0, 0)))
    taps = [((dy % 2) * 2 + (dx % 2), (dy // 2) * W2 + (dx // 2))
            for dy in range(3) for dx in range(3)]
    body = functools.partial(_conv_bn_body, taps=taps, BM=BM, C=C, Wpx=W2,
                             Wvalid=Wo, Mout=Mout)
    y, st = _conv_call(ph, w9, nblk, BM, body)
    return y, st, (Ho, W2, Wo, BM, nblk)


def _dec_conv(act, w9, BM, sig_bias=None):
    N, H, W, C = act.shape
    Hp, Wp = H + 2, W + 2
    Mout = H * Wp
    BM = min(BM, _rup(Mout, 8))
    nblk = _cdiv(Mout, BM)
    R = nblk * BM + 2 * Wp + 10
    xp = jnp.pad(act, ((0, 0), (1, 1), (1, 1), (0, 0)))
    xf = xp.reshape(N, Hp * Wp, C)
    xf = jnp.pad(xf, ((0, 0), (0, R - Hp * Wp), (0, 0)))
    taps = [(0, dy * Wp + dx) for dy in range(3) for dx in range(3)]
    if sig_bias is None:
        body = functools.partial(_conv_bn_body, taps=taps, BM=BM, C=C, Wpx=Wp,
                                 Wvalid=W, Mout=Mout)
        y, st = _conv_call(xf, w9, nblk, BM, body)
        return y, st, (H, Wp, W, BM, nblk)
    body = functools.partial(_conv_sig_body, taps=taps, BM=BM, C=C)
    Co = w9.shape[2]
    MP = nblk * BM
    y = pl.pallas_call(
        body,
        grid=(N, nblk),
        in_specs=[
            pl.BlockSpec((1, R, C), lambda n, b: (n, 0, 0)),
            pl.BlockSpec((9, C, Co), lambda n, b: (0, 0, 0)),
            pl.BlockSpec((8, Co), lambda n, b: (0, 0)),
        ],
        out_specs=pl.BlockSpec((1, BM, Co), lambda n, b: (n, b, 0)),
        out_shape=jax.ShapeDtypeStruct((N, MP, Co), jnp.float32),
    )(xf, w9, sig_bias)
    return y, None, (H, Wp, W, BM, nblk)


def _bn_relu(y, st, gamma, beta, cnt, BM, nblk):
    N, MP, Co = y.shape
    s1, s2 = st[0], st[1]
    mean = s1 / cnt
    var = s2 / cnt - mean * mean
    a = gamma * jax.lax.rsqrt(var + _EPS)
    b = beta - mean * a
    ab = jnp.zeros((8, Co), jnp.float32).at[0].set(a).at[1].set(b)
    return pl.pallas_call(
        _bn_relu_body,
        grid=(N, nblk),
        in_specs=[
            pl.BlockSpec((1, BM, Co), lambda n, b_: (n, b_, 0)),
            pl.BlockSpec((8, Co), lambda n, b_: (0, 0)),
        ],
        out_specs=pl.BlockSpec((1, BM, Co), lambda n, b_: (n, b_, 0)),
        out_shape=jax.ShapeDtypeStruct((N, MP, Co), jnp.float32),
    )(y, ab)


def _extract(t, Ho, Wpx, Wv):
    N = t.shape[0]
    Co = t.shape[2]
    return t[:, :Ho * Wpx].reshape(N, Ho, Wpx, Co)[:, :, :Wv]


def _upsample2x(x):
    n, h, w, c = x.shape

    def idx(s):
        out = 2 * s
        pos = np.arange(out, dtype=np.float64) * ((s - 1) / (out - 1))
        i0 = np.floor(pos).astype(np.int32)
        i1 = np.minimum(i0 + 1, s - 1)
        f = (pos - i0).astype(np.float32)
        return i0, i1, f

    i0, i1, f = idx(h)
    x = x[:, i0] * (1.0 - f)[None, :, None, None] + x[:, i1] * f[None, :, None, None]
    j0, j1, g = idx(w)
    x = x[:, :, j0] * (1.0 - g)[None, None, :, None] + x[:, :, j1] * g[None, None, :, None]
    return x


def _w9(W):
    return jnp.transpose(W, (2, 3, 1, 0)).reshape(9, W.shape[1], W.shape[0])


@jax.jit
def kernel(x, W_enc1, g_enc1, b_enc1, W_enc2, g_enc2, b_enc2, W_enc3, g_enc3,
           b_enc3, W_enc4, g_enc4, b_enc4, W_dec1, g_dec1, b_dec1, W_dec2,
           g_dec2, b_dec2, W_dec3, g_dec3, b_dec3, W_dec4, g_dec4, b_dec4,
           W_out, b_out):
    N = x.shape[0]
    BM = 2048
    act = jnp.transpose(x, (0, 2, 3, 1))
    act = jnp.pad(act, ((0, 0), (0, 0), (0, 0), (0, 7)))
    enc = [(W_enc1, g_enc1, b_enc1), (W_enc2, g_enc2, b_enc2),
           (W_enc3, g_enc3, b_enc3), (W_enc4, g_enc4, b_enc4)]
    for k, (Wc, g_, be_) in enumerate(enc):
        w9 = _w9(Wc)
        if k == 0:
            w9 = jnp.pad(w9, ((0, 0), (0, 7), (0, 0)))
        y, st, (Ho, Wpx, Wv, bm, nblk) = _enc_conv(act, w9, BM)
        cnt = jnp.float32(N * Ho * Wv)
        t = _bn_relu(y, st, g_, be_, cnt, bm, nblk)
        act = _extract(t, Ho, Wpx, Wv)
    latent = jnp.transpose(act, (0, 3, 1, 2))
    dec = [(W_dec1, g_dec1, b_dec1), (W_dec2, g_dec2, b_dec2),
           (W_dec3, g_dec3, b_dec3), (W_dec4, g_dec4, b_dec4)]
    for Wc, g_, be_ in dec:
        up = _upsample2x(act)
        y, st, (Ho, Wpx, Wv, bm, nblk) = _dec_conv(up, _w9(Wc), BM)
        cnt = jnp.float32(N * Ho * Wv)
        t = _bn_relu(y, st, g_, be_, cnt, bm, nblk)
        act = _extract(t, Ho, Wpx, Wv)
    w9o = jnp.pad(_w9(W_out), ((0, 0), (0, 0), (0, 7)))
    bias = jnp.broadcast_to(b_out[0], (8, 8)).astype(jnp.float32)
    y, _, (Ho, Wpx, Wv, bm, nblk) = _dec_conv(act, w9o, BM, sig_bias=bias)
    out = _extract(y, Ho, Wpx, Wv)[..., 0:1]
    out = jnp.transpose(out, (0, 3, 1, 2))
    return (out, latent)
